# baseline (speedup 1.0000x reference)
import functools

import jax
import jax.numpy as jnp
from jax import lax
from jax.experimental import pallas as pl
from jax.experimental.pallas import tpu as pltpu
from jax.experimental.pallas import tpu_sc as plsc


def _sc_gather_body(months_hbm, table_hbm, out_hbm, idx_v, rows_v, sem):
    wid = lax.axis_index("s") * 2 + lax.axis_index("c")

    @pl.when(wid < 6)
    def _():
        base = wid * 8
        pltpu.sync_copy(months_hbm.at[pl.ds(base, 8)], idx_v)
        pltpu.async_copy(table_hbm.at[idx_v], rows_v, sem).wait()
        pltpu.sync_copy(rows_v, out_hbm.at[pl.ds(base, 8), :])


def _sc_gather(months48, month_pad):
    mesh = plsc.VectorSubcoreMesh(core_axis_name="c", subcore_axis_name="s")
    k = functools.partial(
        pl.kernel,
        mesh=mesh,
        out_type=jax.ShapeDtypeStruct((48, 256), jnp.float32),
        scratch_types=[
            pltpu.VMEM((8,), jnp.int32),
            pltpu.VMEM((8, 256), jnp.float32),
            pltpu.SemaphoreType.DMA,
        ],
    )(_sc_gather_body)
    return k(months48, month_pad)


def _body(tokens_ref, mo_ref, ch_ref, pos_ref, out_ref):
    t = 12
    month_e = mo_ref[0][:, :192]
    ch = jnp.broadcast_to(ch_ref[...][None], (t, 3, 192))
    pe = jnp.broadcast_to(pos_ref[:t][:, None], (t, 3, 192))
    me = jnp.broadcast_to(month_e[:, None], (t, 3, 192))
    zero = jnp.zeros((t, 3, 192), jnp.float32)
    addend = jnp.concatenate([ch, pe, me, zero], axis=-1)
    out_ref[...] = tokens_ref[...] + addend[None, None, :, :, None, :]


def kernel(tokens, timestamps, channel_embed, pos_embed, month_table):
    b, h, w, t, bs, d = tokens.shape
    n = d // 4
    months48 = timestamps[:, :, 1].astype(jnp.int32).reshape(b * t)
    month_pad = jnp.pad(month_table, ((0, 0), (0, 64)))
    month_rows = _sc_gather(months48, month_pad).reshape(b, t, 256)
    tok = jnp.transpose(tokens, (0, 1, 3, 4, 2, 5))
    hb = 8
    tok_spec = pl.BlockSpec((1, hb, t, bs, w, d), lambda i, j: (i, j, 0, 0, 0, 0))
    out = pl.pallas_call(
        _body,
        grid=(b, h // hb),
        in_specs=[
            tok_spec,
            pl.BlockSpec((1, t, 256), lambda i, j: (i, 0, 0)),
            pl.BlockSpec((bs, n), lambda i, j: (0, 0)),
            pl.BlockSpec((pos_embed.shape[0], n), lambda i, j: (0, 0)),
        ],
        out_specs=tok_spec,
        out_shape=jax.ShapeDtypeStruct((b, h, t, bs, w, d), tokens.dtype),
    )(tok, month_rows, channel_embed, pos_embed)
    return jnp.transpose(out, (0, 1, 4, 2, 3, 5))


# speedup vs baseline: 1.0149x; 1.0149x over previous
import functools

import jax
import jax.numpy as jnp
from jax import lax
from jax.experimental import pallas as pl
from jax.experimental.pallas import tpu as pltpu
from jax.experimental.pallas import tpu_sc as plsc


def _sc_gather_body(months_hbm, table_hbm, out_hbm, idx_v, rows_v, sem):
    wid = lax.axis_index("s") * 2 + lax.axis_index("c")

    @pl.when(wid < 4)
    def _():
        pltpu.sync_copy(months_hbm.at[wid, 0, :], idx_v)
        pltpu.async_copy(table_hbm.at[idx_v], rows_v, sem).wait()
        pltpu.sync_copy(rows_v, out_hbm.at[wid])


def _sc_gather(months3d, month_pad):
    mesh = plsc.VectorSubcoreMesh(core_axis_name="c", subcore_axis_name="s")
    k = functools.partial(
        pl.kernel,
        mesh=mesh,
        out_type=jax.ShapeDtypeStruct((4, 12, 256), jnp.float32),
        scratch_types=[
            pltpu.VMEM((12,), jnp.int32),
            pltpu.VMEM((12, 256), jnp.float32),
            pltpu.SemaphoreType.DMA,
        ],
    )(_sc_gather_body)
    return k(months3d, month_pad)


def _body(tokens_ref, mo_ref, ch_ref, pos_ref, out_ref):
    t = 12
    month_e = mo_ref[0][:, :192]
    ch = jnp.broadcast_to(ch_ref[...][None], (t, 3, 192))
    pe = jnp.broadcast_to(pos_ref[:t][:, None], (t, 3, 192))
    me = jnp.broadcast_to(month_e[:, None], (t, 3, 192))
    zero = jnp.zeros((t, 3, 192), jnp.float32)
    addend = jnp.concatenate([ch, pe, me, zero], axis=-1)
    out_ref[...] = tokens_ref[...] + addend[None, None, :, :, None, :]


def kernel(tokens, timestamps, channel_embed, pos_embed, month_table):
    b, h, w, t, bs, d = tokens.shape
    n = d // 4
    months3d = timestamps[:, :, 1].astype(jnp.int32).reshape(b, 1, t)
    month_pad = jnp.pad(month_table, ((0, 0), (0, 64)))
    month_rows = _sc_gather(months3d, month_pad)
    tok = jnp.transpose(tokens, (0, 1, 3, 4, 2, 5))
    hb = 8
    tok_spec = pl.BlockSpec((1, hb, t, bs, w, d), lambda i, j: (i, j, 0, 0, 0, 0))
    out = pl.pallas_call(
        _body,
        grid=(b, h // hb),
        in_specs=[
            tok_spec,
            pl.BlockSpec((1, t, 256), lambda i, j: (i, 0, 0)),
            pl.BlockSpec((bs, n), lambda i, j: (0, 0)),
            pl.BlockSpec((pos_embed.shape[0], n), lambda i, j: (0, 0)),
        ],
        out_specs=tok_spec,
        out_shape=jax.ShapeDtypeStruct((b, h, t, bs, w, d), tokens.dtype),
    )(tok, month_rows, channel_embed, pos_embed)
    return jnp.transpose(out, (0, 1, 4, 2, 3, 5))


# speedup vs baseline: 1.2937x; 1.2746x over previous
import jax
import jax.numpy as jnp
from jax.experimental import pallas as pl


def _body(tokens_ref, months_ref, ch_ref, pos_ref, month_ref, out_ref):
    t = 12
    mrow = months_ref[0]
    sel = (jax.lax.broadcasted_iota(jnp.int32, (t, t), 0) == mrow)
    month_e = jax.lax.dot_general(
        sel.astype(jnp.float32), month_ref[...],
        dimension_numbers=(((0,), (0,)), ((), ())),
        preferred_element_type=jnp.float32)
    ch = jnp.broadcast_to(ch_ref[...][None], (t, 3, 192))
    pe = jnp.broadcast_to(pos_ref[:t][:, None], (t, 3, 192))
    me = jnp.broadcast_to(month_e[:, None], (t, 3, 192))
    zero = jnp.zeros((t, 3, 192), jnp.float32)
    addend = jnp.concatenate([ch, pe, me, zero], axis=-1)
    out_ref[...] = tokens_ref[...] + addend[None, None, :, :, None, :]


def kernel(tokens, timestamps, channel_embed, pos_embed, month_table):
    b, h, w, t, bs, d = tokens.shape
    n = d // 4
    months = timestamps[:, :, 1].astype(jnp.int32).reshape(b, 1, t)
    tok = jnp.transpose(tokens, (0, 1, 3, 4, 2, 5))
    hb = 8
    tok_spec = pl.BlockSpec((1, hb, t, bs, w, d), lambda i, j: (i, j, 0, 0, 0, 0))
    out = pl.pallas_call(
        _body,
        grid=(b, h // hb),
        in_specs=[
            tok_spec,
            pl.BlockSpec((1, 1, t), lambda i, j: (i, 0, 0)),
            pl.BlockSpec((bs, n), lambda i, j: (0, 0)),
            pl.BlockSpec((pos_embed.shape[0], n), lambda i, j: (0, 0)),
            pl.BlockSpec((t, n), lambda i, j: (0, 0)),
        ],
        out_specs=tok_spec,
        out_shape=jax.ShapeDtypeStruct((b, h, t, bs, w, d), tokens.dtype),
    )(tok, months, channel_embed, pos_embed, month_table)
    return jnp.transpose(out, (0, 1, 4, 2, 3, 5))
